# baseline (speedup 1.0000x reference)
import functools

import jax
import jax.numpy as jnp
from jax import lax
from jax.experimental import pallas as pl
from jax.experimental.pallas import tpu as pltpu
from jax.experimental.pallas import tpu_sc as plsc

_NC = 2
_NS = 16
_NW = _NC * _NS

_SUB = 128


@functools.lru_cache(maxsize=None)
def _make_lookup(B, H, D):
    mesh = plsc.VectorSubcoreMesh(core_axis_name="c", subcore_axis_name="s")
    bw = B // _NW
    n_sub = bw // _SUB
    T = H * n_sub

    @functools.partial(
        pl.kernel,
        mesh=mesh,
        out_type=jax.ShapeDtypeStruct((H, D, B), jnp.float32),
        scratch_types=[
            pltpu.VMEM((H, bw), jnp.int32),
            pltpu.VMEM((2, _SUB), jnp.int32),
            pltpu.VMEM((2, _SUB, 128), jnp.float32),
            pltpu.VMEM((2, D, _SUB), jnp.float32),
            pltpu.SemaphoreType.DMA((2,)),
            pltpu.SemaphoreType.DMA((2,)),
        ],
        compiler_params=pltpu.CompilerParams(
            use_tc_tiling_on_sc=False, needs_layout_passes=False
        ),
    )
    def lookup_k(idxT_hbm, w128_hbm, outT_hbm, idx_v, g_v, blk_v, out_v,
                 sem_g, sem_o):
        wid = lax.axis_index("s") * _NC + lax.axis_index("c")
        b0 = wid * bw
        pltpu.sync_copy(idxT_hbm.at[:, pl.ds(b0, bw)], idx_v)

        def comp_g(l, s, gb):
            row = idx_v.at[l]
            for j in range(_SUB // 16):
                x = row[pl.ds(s * _SUB + j * 16, 16)]
                g_v[gb, pl.ds(j * 16, 16)] = lax.shift_right_logical(x, 2)

        def fire_gather(gb):
            pltpu.async_copy(w128_hbm.at[g_v.at[gb]], blk_v.at[gb],
                             sem_g.at[gb])

        def wait_gather(gb):
            pltpu.make_async_copy(w128_hbm.at[g_v.at[gb]], blk_v.at[gb],
                                  sem_g.at[gb]).wait()

        def fire_out(l, s, ob):
            pltpu.async_copy(
                out_v.at[ob],
                outT_hbm.at[l, :, pl.ds(b0 + s * _SUB, _SUB)],
                sem_o.at[ob],
            )

        def wait_out(l, s, ob):
            pltpu.make_async_copy(
                out_v.at[ob],
                outT_hbm.at[l, :, pl.ds(b0 + s * _SUB, _SUB)],
                sem_o.at[ob],
            ).wait()

        def extract(l, s, b):
            row = idx_v.at[l]

            def grp_body(j, carry):
                xi = row[pl.ds(s * _SUB + j * 16, 16)]
                colb = lax.bitwise_and(xi, 3) * D
                r16 = lax.iota(jnp.int32, 16) + j * 16
                for d in range(D):
                    v = plsc.load_gather(blk_v.at[b], [r16, colb + d])
                    out_v[b, d, pl.ds(j * 16, 16)] = v
                return carry

            lax.fori_loop(0, _SUB // 16, grp_body, 0)

        def ls(t):
            return t // n_sub, lax.rem(t, n_sub)

        l0, s0 = ls(jnp.int32(0))
        comp_g(l0, s0, 0)
        fire_gather(0)

        def body(u, carry):
            t0 = 2 * u
            t1 = 2 * u + 1
            la, sa = ls(t0)
            lb, sb = ls(t1)
            comp_g(lb, sb, 1)
            fire_gather(1)
            wait_gather(0)

            @pl.when(u >= 1)
            def _():
                lp, sp = ls(t0 - 2)
                wait_out(lp, sp, 0)

            extract(la, sa, 0)
            fire_out(la, sa, 0)
            @pl.when(t0 + 2 < T)
            def _():
                ln, sn = ls(t0 + 2)
                comp_g(ln, sn, 0)
                fire_gather(0)

            wait_gather(1)

            @pl.when(u >= 1)
            def _():
                lp, sp = ls(t1 - 2)
                wait_out(lp, sp, 1)

            extract(lb, sb, 1)
            fire_out(lb, sb, 1)
            return carry

        lax.fori_loop(0, T // 2, body, 0)

        lz0, sz0 = ls(jnp.int32(T - 2))
        wait_out(lz0, sz0, 0)
        lz1, sz1 = ls(jnp.int32(T - 1))
        wait_out(lz1, sz1, 1)

    return lookup_k


def kernel(input, weight):
    B, H = input.shape
    V, D = weight.shape
    idxT = input.T.astype(jnp.int32)
    w128 = weight.reshape(V * D // 128, 128)
    outT = _make_lookup(B, H, D)(idxT, w128)
    return outT.transpose(2, 0, 1)


# speedup vs baseline: 1.0120x; 1.0120x over previous
import functools

import jax
import jax.numpy as jnp
from jax import lax
from jax.experimental import pallas as pl
from jax.experimental.pallas import tpu as pltpu
from jax.experimental.pallas import tpu_sc as plsc

_NC = 2
_NS = 16
_NW = _NC * _NS


@functools.lru_cache(maxsize=None)
def _make_lookup(B, H, D):
    mesh = plsc.VectorSubcoreMesh(core_axis_name="c", subcore_axis_name="s")
    bw = B // _NW

    @functools.partial(
        pl.kernel,
        mesh=mesh,
        out_type=jax.ShapeDtypeStruct((H, D, B), jnp.float32),
        scratch_types=[
            pltpu.VMEM((H, bw), jnp.int32),
            pltpu.VMEM((2, bw, D), jnp.float32),
            pltpu.VMEM((2, D, bw), jnp.float32),
            pltpu.SemaphoreType.DMA((2,)),
            pltpu.SemaphoreType.DMA((2,)),
        ],
        compiler_params=pltpu.CompilerParams(
            use_tc_tiling_on_sc=False, needs_layout_passes=False
        ),
    )
    def lookup_k(idxT_hbm, w_hbm, outT_hbm, idx_v, blk_v, out_v, sem_g, sem_o):
        wid = lax.axis_index("s") * _NC + lax.axis_index("c")
        b0 = wid * bw
        pltpu.sync_copy(idxT_hbm.at[:, pl.ds(b0, bw)], idx_v)

        def fire_gather(l, gb):
            pltpu.async_copy(w_hbm.at[idx_v.at[l]], blk_v.at[gb], sem_g.at[gb])

        def wait_gather(l, gb):
            pltpu.make_async_copy(
                w_hbm.at[idx_v.at[l]], blk_v.at[gb], sem_g.at[gb]
            ).wait()

        def fire_out(l, ob):
            pltpu.async_copy(
                out_v.at[ob], outT_hbm.at[l, :, pl.ds(b0, bw)], sem_o.at[ob]
            )

        def wait_out(l, ob):
            pltpu.make_async_copy(
                out_v.at[ob], outT_hbm.at[l, :, pl.ds(b0, bw)], sem_o.at[ob]
            ).wait()

        def transpose(b):
            def grp_body(j, carry):
                r16 = lax.iota(jnp.int32, 16) + j * 16
                for d in range(D):
                    c16 = jnp.full((16,), d, jnp.int32)
                    v = plsc.load_gather(blk_v.at[b], [r16, c16])
                    out_v[b, d, pl.ds(j * 16, 16)] = v
                return carry

            lax.fori_loop(0, bw // 16, grp_body, 0)

        fire_gather(jnp.int32(0), 0)

        def body(u, carry):
            la = 2 * u
            lb = 2 * u + 1
            fire_gather(lb, 1)
            wait_gather(la, 0)

            @pl.when(u >= 1)
            def _():
                wait_out(la - 2, 0)

            transpose(0)
            fire_out(la, 0)

            @pl.when(lb + 1 < H)
            def _():
                fire_gather(lb + 1, 0)

            wait_gather(lb, 1)

            @pl.when(u >= 1)
            def _():
                wait_out(lb - 2, 1)

            transpose(1)
            fire_out(lb, 1)
            return carry

        lax.fori_loop(0, H // 2, body, 0)

        wait_out(jnp.int32(H - 2), 0)
        wait_out(jnp.int32(H - 1), 1)

    return lookup_k


def kernel(input, weight):
    B, H = input.shape
    V, D = weight.shape
    idxT = input.T.astype(jnp.int32)
    outT = _make_lookup(B, H, D)(idxT, weight)
    return outT.transpose(2, 0, 1)


# speedup vs baseline: 1.6988x; 1.6788x over previous
import functools

import jax
import jax.numpy as jnp
from jax import lax
from jax.experimental import pallas as pl
from jax.experimental.pallas import tpu as pltpu
from jax.experimental.pallas import tpu_sc as plsc

_NC = 2
_NS = 16
_NW = _NC * _NS


@functools.lru_cache(maxsize=None)
def _make_lookup(B, H, D):
    mesh = plsc.VectorSubcoreMesh(core_axis_name="c", subcore_axis_name="s")
    bw = B // _NW

    @functools.partial(
        pl.kernel,
        mesh=mesh,
        out_type=jax.ShapeDtypeStruct((H, D, B), jnp.float32),
        scratch_types=[
            pltpu.VMEM((H, bw), jnp.int32),
            pltpu.VMEM((2, bw, D), jnp.float32),
            pltpu.VMEM((2, D, bw + 1), jnp.float32),
            pltpu.SemaphoreType.DMA((2,)),
            pltpu.SemaphoreType.DMA((2,)),
        ],
        compiler_params=pltpu.CompilerParams(
            use_tc_tiling_on_sc=False, needs_layout_passes=False
        ),
    )
    def lookup_k(idxT_hbm, w_hbm, outT_hbm, idx_v, blk_v, out_v, sem_g, sem_o):
        wid = lax.axis_index("s") * _NC + lax.axis_index("c")
        b0 = wid * bw
        pltpu.sync_copy(idxT_hbm.at[:, pl.ds(b0, bw)], idx_v)

        def fire_gather(l, gb):
            pltpu.async_copy(w_hbm.at[idx_v.at[l]], blk_v.at[gb], sem_g.at[gb])

        def wait_gather(l, gb):
            pltpu.make_async_copy(
                w_hbm.at[idx_v.at[l]], blk_v.at[gb], sem_g.at[gb]
            ).wait()

        def fire_out(l, ob):
            pltpu.async_copy(
                out_v.at[ob, :, pl.ds(0, bw)],
                outT_hbm.at[l, :, pl.ds(b0, bw)],
                sem_o.at[ob],
            )

        def wait_out(l, ob):
            pltpu.make_async_copy(
                out_v.at[ob, :, pl.ds(0, bw)],
                outT_hbm.at[l, :, pl.ds(b0, bw)],
                sem_o.at[ob],
            ).wait()

        d_lo = lax.iota(jnp.int32, 16)
        d_hi = d_lo + 16

        def transpose(b):
            rows = blk_v.at[b]
            outp = out_v.at[b]

            def row_body(k, carry):
                for q in range(4):
                    bp = 4 * k + q
                    bvec = jnp.zeros((16,), jnp.int32) + bp
                    v0 = rows[bp, pl.ds(0, 16)]
                    v1 = rows[bp, pl.ds(16, 16)]
                    plsc.store_scatter(outp, [d_lo, bvec], v0)
                    plsc.store_scatter(outp, [d_hi, bvec], v1)
                return carry

            lax.fori_loop(0, bw // 4, row_body, 0)

        fire_gather(jnp.int32(0), 0)

        def body(u, carry):
            la = 2 * u
            lb = 2 * u + 1
            fire_gather(lb, 1)
            wait_gather(la, 0)

            @pl.when(u >= 1)
            def _():
                wait_out(la - 2, 0)

            transpose(0)
            fire_out(la, 0)

            @pl.when(lb + 1 < H)
            def _():
                fire_gather(lb + 1, 0)

            wait_gather(lb, 1)

            @pl.when(u >= 1)
            def _():
                wait_out(lb - 2, 1)

            transpose(1)
            fire_out(lb, 1)
            return carry

        lax.fori_loop(0, H // 2, body, 0)

        wait_out(jnp.int32(H - 2), 0)
        wait_out(jnp.int32(H - 1), 1)

    return lookup_k


def kernel(input, weight):
    B, H = input.shape
    V, D = weight.shape
    idxT = input.T.astype(jnp.int32)
    outT = _make_lookup(B, H, D)(idxT, weight)
    return outT.transpose(2, 0, 1)
